# baseline (speedup 1.0000x reference)
import jax
import jax.numpy as jnp
from jax.experimental import pallas as pl
from jax.experimental.pallas import tpu as pltpu

_BB = 1
_CB = 64


def _pool_kernel(x_ref, o_ref):
    bb, cb, H, W = x_ref.shape
    x = x_ref[...].reshape(bb * cb, H, W)
    cb = bb * cb
    m1 = jnp.maximum(x, jnp.roll(x, -1, axis=2))
    lane = jax.lax.broadcasted_iota(jnp.int32, (cb, H, W), 2)
    g = jnp.take_along_axis(m1, (2 * lane) % W, axis=2)[:, :, : W // 2]
    m2 = jnp.maximum(g, jnp.roll(g, -1, axis=1))
    v = m2.reshape(cb, H // 16, 2, 8, W // 2)
    tile_even = v[:, :, 0]
    tile_odd = v[:, :, 1]
    s = jax.lax.broadcasted_iota(jnp.int32, tile_even.shape, 2)
    src = (2 * s) % 8
    g_even = jnp.take_along_axis(tile_even, src, axis=2)
    g_odd = jnp.take_along_axis(tile_odd, src, axis=2)
    out = jnp.where(s < 4, g_even, g_odd).reshape(cb, H // 2, W // 2)
    rr = jax.lax.broadcasted_iota(jnp.int32, (cb, cb), 0)
    cc = jax.lax.broadcasted_iota(jnp.int32, (cb, cb), 1)
    eye = (rr == cc).astype(out.dtype)
    out_t = jax.lax.dot_general(
        out,
        eye,
        (((0,), (0,)), ((), ())),
        precision=jax.lax.Precision.HIGHEST,
        preferred_element_type=out.dtype,
    )
    o_ref[...] = out_t[None]


def kernel(x):
    B, C, H, W = x.shape
    grid = (B // _BB, C // _CB)
    out_t = pl.pallas_call(
        _pool_kernel,
        grid=grid,
        in_specs=[pl.BlockSpec((_BB, _CB, H, W), lambda i, j: (i, j, 0, 0))],
        out_specs=pl.BlockSpec((_BB, H // 2, W // 2, _CB), lambda i, j: (i, 0, 0, j)),
        out_shape=jax.ShapeDtypeStruct((B, H // 2, W // 2, C), x.dtype),
        compiler_params=pltpu.CompilerParams(
            dimension_semantics=("parallel", "parallel"),
        ),
    )(x)
    return jnp.transpose(out_t, (0, 3, 1, 2))


# speedup vs baseline: 1.3664x; 1.3664x over previous
import jax
import jax.numpy as jnp
from jax.experimental import pallas as pl
from jax.experimental.pallas import tpu as pltpu

_CB = 64
_CHUNK = 8


def _pool_kernel(x_ref, o_ref, scratch):
    _, cb, H, W = x_ref.shape
    Ho, Wo = H // 2, W // 2
    for ci in range(cb // _CHUNK):
        xc = x_ref[0, ci * _CHUNK:(ci + 1) * _CHUNK]
        m1 = jnp.maximum(xc, jnp.roll(xc, -1, axis=2))
        lane = jax.lax.broadcasted_iota(jnp.int32, m1.shape, 2)
        g = jnp.take_along_axis(m1, (2 * lane) % W, axis=2)[:, :, :Wo]
        m2 = jnp.maximum(g, jnp.roll(g, -1, axis=1))
        v = m2.reshape(_CHUNK, H // 16, 2, 8, Wo)
        tile_even = v[:, :, 0]
        tile_odd = v[:, :, 1]
        s = jax.lax.broadcasted_iota(jnp.int32, tile_even.shape, 2)
        src = (2 * s) % 8
        g_even = jnp.take_along_axis(tile_even, src, axis=2)
        g_odd = jnp.take_along_axis(tile_odd, src, axis=2)
        out_c = jnp.where(s < 4, g_even, g_odd).reshape(_CHUNK, Ho, Wo)
        scratch[ci * _CHUNK:(ci + 1) * _CHUNK] = out_c
    o_ref[...] = jnp.transpose(scratch[...], (1, 2, 0))[None]


def kernel(x):
    B, C, H, W = x.shape
    grid = (B, C // _CB)
    out_t = pl.pallas_call(
        _pool_kernel,
        grid=grid,
        in_specs=[pl.BlockSpec((1, _CB, H, W), lambda i, j: (i, j, 0, 0))],
        out_specs=pl.BlockSpec((1, H // 2, W // 2, _CB), lambda i, j: (i, 0, 0, j)),
        out_shape=jax.ShapeDtypeStruct((B, H // 2, W // 2, C), x.dtype),
        scratch_shapes=[pltpu.VMEM((_CB, H // 2, W // 2), x.dtype)],
        compiler_params=pltpu.CompilerParams(
            dimension_semantics=("parallel", "parallel"),
        ),
    )(x)
    return jnp.transpose(out_t, (0, 3, 1, 2))


# speedup vs baseline: 1.9717x; 1.4429x over previous
import jax
import jax.numpy as jnp
from jax.experimental import pallas as pl
from jax.experimental.pallas import tpu as pltpu

_CB = 64
_CHUNK = 8


def _pool_kernel(x_ref, o_ref, scratch):
    _, cb, H, W = x_ref.shape
    Ho, Wo = H // 2, W // 2
    for ci in range(cb // _CHUNK):
        xc = x_ref[0, ci * _CHUNK:(ci + 1) * _CHUNK]
        m2 = jnp.maximum(xc, jnp.roll(xc, -1, axis=1))
        v = m2.reshape(_CHUNK, H // 16, 2, 8, W)
        tile_even = v[:, :, 0]
        tile_odd = v[:, :, 1]
        s = jax.lax.broadcasted_iota(jnp.int32, tile_even.shape, 2)
        src = (2 * s) % 8
        g_even = jnp.take_along_axis(tile_even, src, axis=2)
        g_odd = jnp.take_along_axis(tile_odd, src, axis=2)
        y = jnp.where(s < 4, g_even, g_odd).reshape(_CHUNK, Ho, W)
        m1 = jnp.maximum(y, jnp.roll(y, -1, axis=2))
        lane = jax.lax.broadcasted_iota(jnp.int32, m1.shape, 2)
        out_c = jnp.take_along_axis(m1, (2 * lane) % W, axis=2)[:, :, :Wo]
        scratch[ci * _CHUNK:(ci + 1) * _CHUNK] = out_c
    o_ref[...] = jnp.transpose(scratch[...], (1, 2, 0))[None]


def kernel(x):
    B, C, H, W = x.shape
    grid = (B, C // _CB)
    out_t = pl.pallas_call(
        _pool_kernel,
        grid=grid,
        in_specs=[pl.BlockSpec((1, _CB, H, W), lambda i, j: (i, j, 0, 0))],
        out_specs=pl.BlockSpec((1, H // 2, W // 2, _CB), lambda i, j: (i, 0, 0, j)),
        out_shape=jax.ShapeDtypeStruct((B, H // 2, W // 2, C), x.dtype),
        scratch_shapes=[pltpu.VMEM((_CB, H // 2, W // 2), x.dtype)],
        compiler_params=pltpu.CompilerParams(
            dimension_semantics=("parallel", "parallel"),
        ),
    )(x)
    return jnp.transpose(out_t, (0, 3, 1, 2))
